# baseline (speedup 1.0000x reference)
import functools
import math

import jax
import jax.numpy as jnp
from jax import lax
from jax.experimental import pallas as pl
from jax.experimental.pallas import tpu as pltpu
from jax.experimental.pallas import tpu_sc as plsc

D_MODEL = 64
SCALE = math.sqrt(D_MODEL)
GROUP = 128
CHUNK_GROUPS = 4


@functools.cache
def _make_sc_lookup(n_groups_total: int):
    info = plsc.get_sparse_core_info()
    nw = info.num_cores * info.num_subcores
    assert n_groups_total % (nw * CHUNK_GROUPS) == 0
    gpw = n_groups_total // nw
    n_chunks = gpw // CHUNK_GROUPS

    mesh = plsc.VectorSubcoreMesh(core_axis_name="c", subcore_axis_name="s")

    @functools.partial(
        pl.kernel,
        out_type=jax.ShapeDtypeStruct((n_groups_total, GROUP, D_MODEL),
                                      jnp.float32),
        mesh=mesh,
        compiler_params=pltpu.CompilerParams(use_tc_tiling_on_sc=False),
        scratch_types=[
            pltpu.VMEM((CHUNK_GROUPS, GROUP), jnp.int32),
            pltpu.VMEM((CHUNK_GROUPS, GROUP, D_MODEL), jnp.float32),
            pltpu.SemaphoreType.DMA,
        ],
    )
    def lookup(x_hbm, lut_hbm, out_hbm, idx_v, rows_v, sem):
        wid = lax.axis_index("s") * info.num_cores + lax.axis_index("c")
        g0 = wid * gpw

        def chunk_body(ci, carry):
            g = g0 + ci * CHUNK_GROUPS
            pltpu.sync_copy(x_hbm.at[pl.ds(g, CHUNK_GROUPS)], idx_v)
            copies = [
                pltpu.async_copy(lut_hbm.at[idx_v.at[j]], rows_v.at[j], sem)
                for j in range(CHUNK_GROUPS)
            ]
            for cp in copies:
                cp.wait()

            def scale_row(r, c2):
                for j in range(CHUNK_GROUPS):
                    for t in range(D_MODEL // 16):
                        v = rows_v[j, r, pl.ds(t * 16, 16)]
                        rows_v[j, r, pl.ds(t * 16, 16)] = v * SCALE
                return c2

            lax.fori_loop(0, GROUP, scale_row, 0)
            pltpu.sync_copy(rows_v, out_hbm.at[pl.ds(g, CHUNK_GROUPS)])
            return carry

        lax.fori_loop(0, n_chunks, chunk_body, 0)

    return lookup


def kernel(x, lut):
    b, s = x.shape
    n = b * s
    x2 = x.reshape(n // GROUP, GROUP).astype(jnp.int32)
    out = _make_sc_lookup(n // GROUP)(x2, lut)
    return out.reshape(b, s, D_MODEL)


# speedup vs baseline: 1.0915x; 1.0915x over previous
import functools
import math

import jax
import jax.numpy as jnp
from jax import lax
from jax.experimental import pallas as pl
from jax.experimental.pallas import tpu as pltpu
from jax.experimental.pallas import tpu_sc as plsc

D_MODEL = 64
SCALE = math.sqrt(D_MODEL)
GROUP = 128
CHUNK_GROUPS = 4


@functools.cache
def _make_sc_lookup(n_groups_total: int):
    info = plsc.get_sparse_core_info()
    nw = info.num_cores * info.num_subcores
    assert n_groups_total % (nw * CHUNK_GROUPS) == 0
    gpw = n_groups_total // nw
    n_chunks = gpw // CHUNK_GROUPS

    mesh = plsc.VectorSubcoreMesh(core_axis_name="c", subcore_axis_name="s")

    @functools.partial(
        pl.kernel,
        out_type=jax.ShapeDtypeStruct((n_groups_total, GROUP, D_MODEL),
                                      jnp.float32),
        mesh=mesh,
        compiler_params=pltpu.CompilerParams(use_tc_tiling_on_sc=False),
        scratch_types=[
            pltpu.VMEM((gpw, GROUP), jnp.int32),
            pltpu.VMEM((2, CHUNK_GROUPS, GROUP, D_MODEL), jnp.float32),
            pltpu.SemaphoreType.DMA,
            pltpu.SemaphoreType.DMA,
        ],
    )
    def lookup(x_hbm, lut_hbm, out_hbm, idx_v, rows_v, gsem, wsem):
        wid = lax.axis_index("s") * info.num_cores + lax.axis_index("c")
        g0 = wid * gpw
        pltpu.sync_copy(x_hbm.at[pl.ds(g0, gpw)], idx_v)

        def fire_gathers(ci, buf):
            for j in range(CHUNK_GROUPS):
                pltpu.async_copy(
                    lut_hbm.at[idx_v.at[ci * CHUNK_GROUPS + j]],
                    rows_v.at[buf, j], gsem)

        def wait_gathers():
            for j in range(CHUNK_GROUPS):
                pltpu.make_async_copy(
                    lut_hbm.at[pl.ds(0, GROUP)], rows_v.at[0, j], gsem).wait()

        def fire_write(ci, buf):
            pltpu.async_copy(
                rows_v.at[buf],
                out_hbm.at[pl.ds(g0 + ci * CHUNK_GROUPS, CHUNK_GROUPS)], wsem)

        def wait_write():
            pltpu.make_async_copy(
                rows_v.at[0], out_hbm.at[pl.ds(0, CHUNK_GROUPS)], wsem).wait()

        fire_gathers(0, 0)

        def chunk_body(ci, carry):
            buf = ci & 1
            wait_gathers()

            @pl.when(ci >= 1)
            def _():
                wait_write()

            @pl.when(ci + 1 < n_chunks)
            def _():
                fire_gathers(ci + 1, 1 - buf)

            @plsc.parallel_loop(0, CHUNK_GROUPS * GROUP, unroll=2)
            def _scale(i):
                j = i >> 7
                r = i & (GROUP - 1)
                for t in range(D_MODEL // 16):
                    v = rows_v[buf, j, r, pl.ds(t * 16, 16)]
                    rows_v[buf, j, r, pl.ds(t * 16, 16)] = v * SCALE

            fire_write(ci, buf)
            return carry

        lax.fori_loop(0, n_chunks, chunk_body, 0)
        wait_write()

    return lookup


def kernel(x, lut):
    b, s = x.shape
    n = b * s
    x2 = x.reshape(n // GROUP, GROUP).astype(jnp.int32)
    out = _make_sc_lookup(n // GROUP)(x2, lut)
    return out.reshape(b, s, D_MODEL)


# speedup vs baseline: 1.2652x; 1.1591x over previous
import functools
import math

import jax
import jax.numpy as jnp
from jax import lax
from jax.experimental import pallas as pl
from jax.experimental.pallas import tpu as pltpu
from jax.experimental.pallas import tpu_sc as plsc

D_MODEL = 64
D_PAD = 128
SCALE = math.sqrt(D_MODEL)
GROUP = 128


@functools.cache
def _make_sc_lookup(n: int):
    info = plsc.get_sparse_core_info()
    nw = info.num_cores * info.num_subcores
    assert n % (nw * GROUP) == 0
    ipw = n // nw
    n_groups = ipw // GROUP

    mesh = plsc.VectorSubcoreMesh(core_axis_name="c", subcore_axis_name="s")

    @functools.partial(
        pl.kernel,
        out_type=jax.ShapeDtypeStruct((n, D_MODEL), jnp.float32),
        mesh=mesh,
        scratch_types=[
            pltpu.VMEM((ipw,), jnp.int32),
            pltpu.VMEM((2, GROUP, D_PAD), jnp.float32),
            pltpu.VMEM((2, GROUP, D_MODEL), jnp.float32),
            pltpu.SemaphoreType.DMA,
            pltpu.SemaphoreType.DMA,
        ],
    )
    def lookup(x_hbm, lut_hbm, out_hbm, idx_v, grows_v, orows_v, gsem, wsem):
        wid = lax.axis_index("s") * info.num_cores + lax.axis_index("c")
        i0 = wid * ipw
        pltpu.sync_copy(x_hbm.at[pl.ds(i0, ipw)], idx_v)

        def fire_gather(gi, buf):
            off = pl.multiple_of(gi * GROUP, GROUP)
            pltpu.async_copy(
                lut_hbm.at[idx_v.at[pl.ds(off, GROUP)]],
                grows_v.at[buf], gsem)

        def wait_gather():
            pltpu.make_async_copy(
                lut_hbm.at[pl.ds(0, GROUP)], grows_v.at[0], gsem).wait()

        def fire_write(gi, buf):
            off = pl.multiple_of(i0 + gi * GROUP, GROUP)
            pltpu.async_copy(
                orows_v.at[buf], out_hbm.at[pl.ds(off, GROUP)], wsem)

        def wait_write():
            pltpu.make_async_copy(
                orows_v.at[0], out_hbm.at[pl.ds(0, GROUP)], wsem).wait()

        fire_gather(0, 0)

        def group_body(gi, carry):
            buf = gi & 1
            wait_gather()

            @pl.when(gi >= 1)
            def _():
                wait_write()

            @pl.when(gi + 1 < n_groups)
            def _():
                fire_gather(gi + 1, 1 - buf)

            @plsc.parallel_loop(0, GROUP, unroll=2)
            def _scale(r):
                for t in range(D_MODEL // 16):
                    v = grows_v[buf, r, pl.ds(t * 16, 16)]
                    orows_v[buf, r, pl.ds(t * 16, 16)] = v * SCALE

            fire_write(gi, buf)
            return carry

        lax.fori_loop(0, n_groups, group_body, 0)
        wait_write()

    return lookup


def kernel(x, lut):
    b, s = x.shape
    n = b * s
    x1 = x.reshape(-1).astype(jnp.int32)
    lut_p = jnp.pad(lut, ((0, 0), (0, D_PAD - D_MODEL)))
    out = _make_sc_lookup(n)(x1, lut_p)
    return out.reshape(b, s, D_MODEL)
